# Initial kernel scaffold; baseline (speedup 1.0000x reference)
import jax
import jax.numpy as jnp
from jax.experimental import pallas as pl


def kernel(x, edge_index_0, edge_index_1, Wl0, bl0, Wr0, Wl1, bl1, Wr1):
    raise NotImplementedError("write your pallas kernel here")


# speedup vs baseline: 9.9362x; 9.9362x over previous
import functools

import jax
import jax.numpy as jnp
from jax import lax
from jax.experimental import pallas as pl
from jax.experimental.pallas import tpu as pltpu
from jax.experimental.pallas import tpu_sc as plsc

N = 10000
E = 320000
D = 128
H = 16
C = 14

NC = 2
NS = 16
NW = NC * NS
CHUNK = 128
N_PAD = 10240
E_PAD = 327680
K = E_PAD // (NW * CHUNK)


def _segment_sum_sc(table, src, dst, w):
    rpt = N_PAD // NS

    mesh = plsc.VectorSubcoreMesh(core_axis_name="c", subcore_axis_name="s")

    @functools.partial(
        pl.kernel,
        mesh=mesh,
        compiler_params=pltpu.CompilerParams(use_tc_tiling_on_sc=False),
        out_type=jax.ShapeDtypeStruct((NC, N_PAD, w), jnp.float32),
        scratch_types=[
            pltpu.VMEM((K, CHUNK), jnp.int32),
            pltpu.VMEM((K, CHUNK), jnp.int32),
            pltpu.VMEM((CHUNK, w), jnp.float32),
            pltpu.VMEM((rpt, w), jnp.float32),
            pltpu.VMEM_SHARED((N_PAD, w), jnp.float32),
            pltpu.SemaphoreType.DMA,
        ],
    )
    def k(table_hbm, src_hbm, dst_hbm, out_hbm, src_v, dst_v, rows_v, io_v,
          acc_s, sem):
        c = lax.axis_index("c")
        s = lax.axis_index("s")
        wid = s * NC + c

        pltpu.sync_copy(src_hbm.at[wid], src_v)
        pltpu.sync_copy(dst_hbm.at[wid], dst_v)

        def zero_body(i, _):
            io_v[pl.ds(i * 16, 16), :] = jnp.zeros((16, w), jnp.float32)
            return 0

        lax.fori_loop(0, rpt // 16, zero_body, 0)
        pltpu.sync_copy(io_v, acc_s.at[pl.ds(s * rpt, rpt)])
        plsc.subcore_barrier()

        def body(j, _):
            pltpu.async_copy(table_hbm.at[src_v.at[j]], rows_v, sem).wait()
            pltpu.sync_copy(rows_v, acc_s.at[dst_v.at[j]], add=True)
            return 0

        lax.fori_loop(0, K, body, 0)
        plsc.subcore_barrier()

        pltpu.sync_copy(acc_s.at[pl.ds(s * rpt, rpt)],
                        out_hbm.at[c, pl.ds(s * rpt, rpt)])

    return k(table, src, dst)


def _proj0_tc(xp, wl0t_pad, wr0t):
    def body(x_ref, wl_ref, wr_ref, t0_ref, z0_ref):
        xv = x_ref[...]
        y = jnp.dot(xv, wl_ref[...], preferred_element_type=jnp.float32)
        col = lax.broadcasted_iota(jnp.int32, (1, 32), 1)
        t0_ref[...] = y + jnp.where(col == H, 1.0, 0.0)
        z0_ref[...] = jnp.dot(xv, wr_ref[...],
                              preferred_element_type=jnp.float32)

    return pl.pallas_call(
        body,
        out_shape=(
            jax.ShapeDtypeStruct((N_PAD, 32), jnp.float32),
            jax.ShapeDtypeStruct((N_PAD, H), jnp.float32),
        ),
    )(xp, wl0t_pad, wr0t)


def _mid_tc(p0, z0, bl0_row, wl1t_pad, wr1t_pad, bl1_row):
    def body(p_ref, z0_ref, bl0_ref, wl_ref, wr_ref, bl1_ref, t1_ref, z1_ref):
        ssum = p_ref[0] + p_ref[1]
        cnt = jnp.maximum(ssum[:, H:H + 1], 1.0)
        agg = ssum[:, :H] * (1.0 / cnt)
        h = jnp.maximum(agg + bl0_ref[...] + z0_ref[...], 0.0)
        y = jnp.dot(h, wl_ref[...], preferred_element_type=jnp.float32)
        col = lax.broadcasted_iota(jnp.int32, (1, 16), 1)
        t1_ref[...] = y + jnp.where(col == C, 1.0, 0.0)
        z1_ref[...] = jnp.dot(h, wr_ref[...],
                              preferred_element_type=jnp.float32) + bl1_ref[...]

    return pl.pallas_call(
        body,
        out_shape=(
            jax.ShapeDtypeStruct((N_PAD, 16), jnp.float32),
            jax.ShapeDtypeStruct((N_PAD, 16), jnp.float32),
        ),
    )(p0, z0, bl0_row, wl1t_pad, wr1t_pad, bl1_row)


def _final_tc(p1, z1):
    def body(p_ref, z1_ref, o_ref):
        ssum = p_ref[0] + p_ref[1]
        cnt = jnp.maximum(ssum[:, C:C + 1], 1.0)
        o = ssum * (1.0 / cnt) + z1_ref[...]
        col = lax.broadcasted_iota(jnp.int32, (1, 16), 1)
        msk = col < C
        om = jnp.where(msk, o, -jnp.inf)
        m = jnp.max(om, axis=1, keepdims=True)
        e = jnp.where(msk, jnp.exp(om - m), 0.0)
        lse = jnp.log(jnp.sum(e, axis=1, keepdims=True))
        o_ref[...] = o - m - lse

    return pl.pallas_call(
        body,
        out_shape=jax.ShapeDtypeStruct((N_PAD, 16), jnp.float32),
    )(p1, z1)


def _pad_edges(edge_index):
    src = edge_index[0]
    dst = edge_index[1]
    pad = E_PAD - E
    src = jnp.concatenate([src, jnp.zeros((pad,), jnp.int32)])
    dst = jnp.concatenate([dst, jnp.full((pad,), N, jnp.int32)])
    return (src.reshape(NW, K, CHUNK), dst.reshape(NW, K, CHUNK))


def kernel(x, edge_index_0, edge_index_1, Wl0, bl0, Wr0, Wl1, bl1, Wr1):
    xp = jnp.pad(x, ((0, N_PAD - N), (0, 0)))
    src0, dst0 = _pad_edges(edge_index_0)
    src1, dst1 = _pad_edges(edge_index_1)

    wl0t_pad = jnp.pad(Wl0.T, ((0, 0), (0, 32 - H)))
    wr0t = Wr0.T
    wl1t_pad = jnp.pad(Wl1.T, ((0, 0), (0, 16 - C)))
    wr1t_pad = jnp.pad(Wr1.T, ((0, 0), (0, 16 - C)))
    bl0_row = bl0.reshape(1, H)
    bl1_row = jnp.pad(bl1, (0, 16 - C)).reshape(1, 16)

    table0, z0 = _proj0_tc(xp, wl0t_pad, wr0t)
    p0 = _segment_sum_sc(table0, src0, dst0, 32)
    table1, z1 = _mid_tc(p0, z0, bl0_row, wl1t_pad, wr1t_pad, bl1_row)
    p1 = _segment_sum_sc(table1, src1, dst1, 16)
    out = _final_tc(p1, z1)
    return out[:N, :C]


# speedup vs baseline: 13.1958x; 1.3280x over previous
import functools

import jax
import jax.numpy as jnp
from jax import lax
from jax.experimental import pallas as pl
from jax.experimental.pallas import tpu as pltpu
from jax.experimental.pallas import tpu_sc as plsc

N = 10000
E = 320000
D = 128
H = 16
C = 14

NC = 2
NS = 16
NW = NC * NS
CHUNK = 128
N_PAD = 10240
E_PAD = 327680
K = E_PAD // (NW * CHUNK)
NBUF = 4


def _segment_sum_sc(table, src, dst, w):
    rpt = N_PAD // NS

    mesh = plsc.VectorSubcoreMesh(core_axis_name="c", subcore_axis_name="s")

    @functools.partial(
        pl.kernel,
        mesh=mesh,
        compiler_params=pltpu.CompilerParams(use_tc_tiling_on_sc=False),
        out_type=jax.ShapeDtypeStruct((NC, N_PAD, w), jnp.float32),
        scratch_types=[
            pltpu.VMEM((K, CHUNK), jnp.int32),
            pltpu.VMEM((K, CHUNK), jnp.int32),
            [pltpu.VMEM((CHUNK, w), jnp.float32) for _ in range(NBUF)],
            pltpu.VMEM((rpt, w), jnp.float32),
            pltpu.VMEM_SHARED((N_PAD, w), jnp.float32),
            [pltpu.SemaphoreType.DMA for _ in range(NBUF)],
        ],
    )
    def k(table_hbm, src_hbm, dst_hbm, out_hbm, src_v, dst_v, rows_v, io_v,
          acc_s, sems):
        c = lax.axis_index("c")
        s = lax.axis_index("s")
        wid = s * NC + c

        pltpu.sync_copy(src_hbm.at[wid], src_v)
        pltpu.sync_copy(dst_hbm.at[wid], dst_v)

        def zero_body(i, _):
            io_v[pl.ds(i * 16, 16), :] = jnp.zeros((16, w), jnp.float32)
            return 0

        lax.fori_loop(0, rpt // 16, zero_body, 0)
        pltpu.sync_copy(io_v, acc_s.at[pl.ds(s * rpt, rpt)])
        plsc.subcore_barrier()

        for r in range(NBUF):
            pltpu.async_copy(table_hbm.at[src_v.at[r]], rows_v[r], sems[r])

        def body(i, _):
            for r in range(NBUF):
                j = i * NBUF + r
                pltpu.make_async_copy(
                    table_hbm.at[src_v.at[j]], rows_v[r], sems[r]).wait()
                pltpu.sync_copy(rows_v[r], acc_s.at[dst_v.at[j]], add=True)

                @pl.when(j + NBUF < K)
                def _():
                    pltpu.async_copy(
                        table_hbm.at[src_v.at[j + NBUF]], rows_v[r], sems[r])
            return 0

        lax.fori_loop(0, K // NBUF, body, 0)
        plsc.subcore_barrier()

        pltpu.sync_copy(acc_s.at[pl.ds(s * rpt, rpt)],
                        out_hbm.at[c, pl.ds(s * rpt, rpt)])

    return k(table, src, dst)


def _proj0_tc(xp, wl0t_pad, wr0t):
    def body(x_ref, wl_ref, wr_ref, t0_ref, z0_ref):
        xv = x_ref[...]
        y = jnp.dot(xv, wl_ref[...], preferred_element_type=jnp.float32)
        col = lax.broadcasted_iota(jnp.int32, (1, 32), 1)
        t0_ref[...] = y + jnp.where(col == H, 1.0, 0.0)
        z0_ref[...] = jnp.dot(xv, wr_ref[...],
                              preferred_element_type=jnp.float32)

    return pl.pallas_call(
        body,
        out_shape=(
            jax.ShapeDtypeStruct((N_PAD, 32), jnp.float32),
            jax.ShapeDtypeStruct((N_PAD, H), jnp.float32),
        ),
    )(xp, wl0t_pad, wr0t)


def _mid_tc(p0, z0, bl0_row, wl1t_pad, wr1t_pad, bl1_row):
    def body(p_ref, z0_ref, bl0_ref, wl_ref, wr_ref, bl1_ref, t1_ref, z1_ref):
        ssum = p_ref[0] + p_ref[1]
        cnt = jnp.maximum(ssum[:, H:H + 1], 1.0)
        agg = ssum[:, :H] * (1.0 / cnt)
        h = jnp.maximum(agg + bl0_ref[...] + z0_ref[...], 0.0)
        y = jnp.dot(h, wl_ref[...], preferred_element_type=jnp.float32)
        col = lax.broadcasted_iota(jnp.int32, (1, 16), 1)
        t1_ref[...] = y + jnp.where(col == C, 1.0, 0.0)
        z1_ref[...] = jnp.dot(h, wr_ref[...],
                              preferred_element_type=jnp.float32) + bl1_ref[...]

    return pl.pallas_call(
        body,
        out_shape=(
            jax.ShapeDtypeStruct((N_PAD, 16), jnp.float32),
            jax.ShapeDtypeStruct((N_PAD, 16), jnp.float32),
        ),
    )(p0, z0, bl0_row, wl1t_pad, wr1t_pad, bl1_row)


def _final_tc(p1, z1):
    def body(p_ref, z1_ref, o_ref):
        ssum = p_ref[0] + p_ref[1]
        cnt = jnp.maximum(ssum[:, C:C + 1], 1.0)
        o = ssum * (1.0 / cnt) + z1_ref[...]
        col = lax.broadcasted_iota(jnp.int32, (1, 16), 1)
        msk = col < C
        om = jnp.where(msk, o, -jnp.inf)
        m = jnp.max(om, axis=1, keepdims=True)
        e = jnp.where(msk, jnp.exp(om - m), 0.0)
        lse = jnp.log(jnp.sum(e, axis=1, keepdims=True))
        o_ref[...] = o - m - lse

    return pl.pallas_call(
        body,
        out_shape=jax.ShapeDtypeStruct((N_PAD, 16), jnp.float32),
    )(p1, z1)


def _pad_edges(edge_index):
    src = edge_index[0]
    dst = edge_index[1]
    pad = E_PAD - E
    src = jnp.concatenate([src, jnp.zeros((pad,), jnp.int32)])
    dst = jnp.concatenate([dst, jnp.full((pad,), N, jnp.int32)])
    return (src.reshape(NW, K, CHUNK), dst.reshape(NW, K, CHUNK))


def kernel(x, edge_index_0, edge_index_1, Wl0, bl0, Wr0, Wl1, bl1, Wr1):
    xp = jnp.pad(x, ((0, N_PAD - N), (0, 0)))
    src0, dst0 = _pad_edges(edge_index_0)
    src1, dst1 = _pad_edges(edge_index_1)

    wl0t_pad = jnp.pad(Wl0.T, ((0, 0), (0, 32 - H)))
    wr0t = Wr0.T
    wl1t_pad = jnp.pad(Wl1.T, ((0, 0), (0, 16 - C)))
    wr1t_pad = jnp.pad(Wr1.T, ((0, 0), (0, 16 - C)))
    bl0_row = bl0.reshape(1, H)
    bl1_row = jnp.pad(bl1, (0, 16 - C)).reshape(1, 16)

    table0, z0 = _proj0_tc(xp, wl0t_pad, wr0t)
    p0 = _segment_sum_sc(table0, src0, dst0, 32)
    table1, z1 = _mid_tc(p0, z0, bl0_row, wl1t_pad, wr1t_pad, bl1_row)
    p1 = _segment_sum_sc(table1, src1, dst1, 16)
    out = _final_tc(p1, z1)
    return out[:N, :C]


# speedup vs baseline: 14.1609x; 1.0731x over previous
import functools

import jax
import jax.numpy as jnp
from jax import lax
from jax.experimental import pallas as pl
from jax.experimental.pallas import tpu as pltpu
from jax.experimental.pallas import tpu_sc as plsc

N = 10000
E = 320000
D = 128
H = 16
C = 14

NC = 2
NS = 16
NW = NC * NS
CHUNK = 128
N_PAD = 10240
E_PAD = 327680
K = E_PAD // (NW * CHUNK)
NBUF = 4


def _segment_sum_sc(table, src, dst, w, with_hist):
    rpt = N_PAD // NS

    mesh = plsc.VectorSubcoreMesh(core_axis_name="c", subcore_axis_name="s")

    out_type = [jax.ShapeDtypeStruct((NC, N_PAD, w), jnp.float32)]
    scratch = [
        pltpu.VMEM((K, CHUNK), jnp.int32),
        pltpu.VMEM((K, CHUNK), jnp.int32),
        [pltpu.VMEM((CHUNK, w), jnp.float32) for _ in range(NBUF)],
        pltpu.VMEM_SHARED((N_PAD, w), jnp.float32),
        [pltpu.SemaphoreType.DMA for _ in range(NBUF)],
    ]
    if with_hist:
        out_type.append(jax.ShapeDtypeStruct((NC, N_PAD), jnp.float32))
        scratch += [
            pltpu.VMEM((N_PAD,), jnp.float32),
            pltpu.VMEM_SHARED((NS, N_PAD), jnp.float32),
            pltpu.VMEM((NS * rpt,), jnp.float32),
            pltpu.VMEM((rpt,), jnp.float32),
        ]
    zeros = jnp.zeros((N_PAD, w), jnp.float32)

    @functools.partial(
        pl.kernel,
        mesh=mesh,
        compiler_params=pltpu.CompilerParams(
            use_tc_tiling_on_sc=False,
            needs_layout_passes=not with_hist,
        ),
        out_type=out_type,
        scratch_types=scratch,
    )
    def k(*refs):
        if with_hist:
            (table_hbm, src_hbm, dst_hbm, zeros_hbm, out_hbm,
             cnt_hbm, src_v, dst_v, rows_v, acc_s, sems,
             hist_v, stage_s, red_v, csum_v) = refs
        else:
            (table_hbm, src_hbm, dst_hbm, zeros_hbm, out_hbm,
             src_v, dst_v, rows_v, acc_s, sems) = refs
        c = lax.axis_index("c")
        s = lax.axis_index("s")
        wid = s * NC + c

        pltpu.sync_copy(src_hbm.at[wid], src_v)
        pltpu.sync_copy(dst_hbm.at[wid], dst_v)

        pltpu.sync_copy(zeros_hbm.at[pl.ds(s * rpt, rpt)],
                        acc_s.at[pl.ds(s * rpt, rpt)])

        if with_hist:
            def zero_hist(i, _):
                hist_v[pl.ds(i * 16, 16)] = jnp.zeros((16,), jnp.float32)
                return 0

            lax.fori_loop(0, N_PAD // 16, zero_hist, 0)

        plsc.subcore_barrier()

        for r in range(NBUF):
            pltpu.async_copy(table_hbm.at[src_v.at[r]], rows_v[r], sems[r])

        ones16 = jnp.ones((16,), jnp.float32)

        def body(i, _):
            for r in range(NBUF):
                j = i * NBUF + r
                pltpu.make_async_copy(
                    table_hbm.at[src_v.at[j]], rows_v[r], sems[r]).wait()
                pltpu.sync_copy(rows_v[r], acc_s.at[dst_v.at[j]], add=True)
                if with_hist:
                    for q in range(CHUNK // 16):
                        idx = dst_v[j, pl.ds(q * 16, 16)]
                        plsc.addupdate_scatter(hist_v, [idx], ones16)

                @pl.when(j + NBUF < K)
                def _():
                    pltpu.async_copy(
                        table_hbm.at[src_v.at[j + NBUF]], rows_v[r], sems[r])
            return 0

        lax.fori_loop(0, K // NBUF, body, 0)
        if with_hist:
            pltpu.sync_copy(hist_v, stage_s.at[s])
        plsc.subcore_barrier()

        pltpu.sync_copy(acc_s.at[pl.ds(s * rpt, rpt)],
                        out_hbm.at[c, pl.ds(s * rpt, rpt)])

        if with_hist:
            for r in range(NS):
                pltpu.sync_copy(stage_s.at[r, pl.ds(s * rpt, rpt)],
                                red_v.at[pl.ds(r * rpt, rpt)])

            def red_body(i, _):
                acc = red_v[pl.ds(i * 16, 16)]
                for r in range(1, NS):
                    acc = acc + red_v[pl.ds(r * rpt + i * 16, 16)]
                csum_v[pl.ds(i * 16, 16)] = acc
                return 0

            lax.fori_loop(0, rpt // 16, red_body, 0)
            pltpu.sync_copy(csum_v, cnt_hbm.at[c, pl.ds(s * rpt, rpt)])

    return k(table, src, dst, zeros)


def _proj0_tc(xp, wl0t, wr0t):
    def body(x_ref, wl_ref, wr_ref, t0_ref, z0_ref):
        xv = x_ref[...]
        t0_ref[...] = jnp.dot(xv, wl_ref[...],
                              preferred_element_type=jnp.float32)
        z0_ref[...] = jnp.dot(xv, wr_ref[...],
                              preferred_element_type=jnp.float32)

    return pl.pallas_call(
        body,
        out_shape=(
            jax.ShapeDtypeStruct((N_PAD, H), jnp.float32),
            jax.ShapeDtypeStruct((N_PAD, H), jnp.float32),
        ),
    )(xp, wl0t, wr0t)


def _mid_tc(p0, c0, z0, bl0_row, wl1t_pad, wr1t_pad, bl1_row):
    def body(p_ref, c_ref, z0_ref, bl0_ref, wl_ref, wr_ref, bl1_ref,
             t1_ref, z1_ref):
        ssum = p_ref[0] + p_ref[1]
        cnt = jnp.maximum(c_ref[0] + c_ref[1], 1.0)
        agg = ssum * (1.0 / cnt)
        h = jnp.maximum(agg + bl0_ref[...] + z0_ref[...], 0.0)
        y = jnp.dot(h, wl_ref[...], preferred_element_type=jnp.float32)
        col = lax.broadcasted_iota(jnp.int32, (1, 16), 1)
        t1_ref[...] = y + jnp.where(col == C, 1.0, 0.0)
        z1_ref[...] = jnp.dot(h, wr_ref[...],
                              preferred_element_type=jnp.float32) + bl1_ref[...]

    return pl.pallas_call(
        body,
        out_shape=(
            jax.ShapeDtypeStruct((N_PAD, 16), jnp.float32),
            jax.ShapeDtypeStruct((N_PAD, 16), jnp.float32),
        ),
    )(p0, c0, z0, bl0_row, wl1t_pad, wr1t_pad, bl1_row)


def _final_tc(p1, z1):
    def body(p_ref, z1_ref, o_ref):
        ssum = p_ref[0] + p_ref[1]
        cnt = jnp.maximum(ssum[:, C:C + 1], 1.0)
        o = ssum * (1.0 / cnt) + z1_ref[...]
        col = lax.broadcasted_iota(jnp.int32, (1, 16), 1)
        msk = col < C
        om = jnp.where(msk, o, -jnp.inf)
        m = jnp.max(om, axis=1, keepdims=True)
        e = jnp.where(msk, jnp.exp(om - m), 0.0)
        lse = jnp.log(jnp.sum(e, axis=1, keepdims=True))
        o_ref[...] = o - m - lse

    return pl.pallas_call(
        body,
        out_shape=jax.ShapeDtypeStruct((N_PAD, 16), jnp.float32),
    )(p1, z1)


def _pad_edges(edge_index):
    src = edge_index[0]
    dst = edge_index[1]
    pad = E_PAD - E
    src = jnp.concatenate([src, jnp.zeros((pad,), jnp.int32)])
    dst = jnp.concatenate([dst, jnp.full((pad,), N, jnp.int32)])
    return (src.reshape(NW, K, CHUNK), dst.reshape(NW, K, CHUNK))


def kernel(x, edge_index_0, edge_index_1, Wl0, bl0, Wr0, Wl1, bl1, Wr1):
    xp = jnp.pad(x, ((0, N_PAD - N), (0, 0)))
    src0, dst0 = _pad_edges(edge_index_0)
    src1, dst1 = _pad_edges(edge_index_1)

    wl0t = Wl0.T
    wr0t = Wr0.T
    wl1t_pad = jnp.pad(Wl1.T, ((0, 0), (0, 16 - C)))
    wr1t_pad = jnp.pad(Wr1.T, ((0, 0), (0, 16 - C)))
    bl0_row = bl0.reshape(1, H)
    bl1_row = jnp.pad(bl1, (0, 16 - C)).reshape(1, 16)

    table0, z0 = _proj0_tc(xp, wl0t, wr0t)
    p0, c0 = _segment_sum_sc(table0, src0, dst0, H, True)
    table1, z1 = _mid_tc(p0, c0.reshape(NC, N_PAD, 1), z0, bl0_row,
                         wl1t_pad, wr1t_pad, bl1_row)
    p1, = _segment_sum_sc(table1, src1, dst1, 16, False)
    out = _final_tc(p1, z1)
    return out[:N, :C]


# speedup vs baseline: 14.1726x; 1.0008x over previous
import functools

import jax
import jax.numpy as jnp
from jax import lax
from jax.experimental import pallas as pl
from jax.experimental.pallas import tpu as pltpu
from jax.experimental.pallas import tpu_sc as plsc

N = 10000
E = 320000
D = 128
H = 16
C = 14

NC = 2
NS = 16
NW = NC * NS
CHUNK = 128
N_PAD = 10240
E_PAD = 327680
K = E_PAD // (NW * CHUNK)
NBUF = 8
GLA = 4


def _segment_sum_sc(table, src, dst, w, with_hist):
    rpt = N_PAD // NS

    mesh = plsc.VectorSubcoreMesh(core_axis_name="c", subcore_axis_name="s")

    out_type = [jax.ShapeDtypeStruct((NC, N_PAD, w), jnp.float32)]
    scratch = [
        pltpu.VMEM((K, CHUNK), jnp.int32),
        pltpu.VMEM((K, CHUNK), jnp.int32),
        [pltpu.VMEM((CHUNK, w), jnp.float32) for _ in range(NBUF)],
        pltpu.VMEM_SHARED((N_PAD, w), jnp.float32),
        [pltpu.SemaphoreType.DMA for _ in range(NBUF)],
        [pltpu.SemaphoreType.DMA for _ in range(NBUF)],
    ]
    if with_hist:
        out_type.append(jax.ShapeDtypeStruct((NC, N_PAD), jnp.float32))
        scratch += [
            pltpu.VMEM((N_PAD,), jnp.float32),
            pltpu.VMEM_SHARED((NS, N_PAD), jnp.float32),
            pltpu.VMEM((NS * rpt,), jnp.float32),
            pltpu.VMEM((rpt,), jnp.float32),
        ]
    zeros = jnp.zeros((N_PAD, w), jnp.float32)

    @functools.partial(
        pl.kernel,
        mesh=mesh,
        compiler_params=pltpu.CompilerParams(
            use_tc_tiling_on_sc=False,
            needs_layout_passes=not with_hist,
        ),
        out_type=out_type,
        scratch_types=scratch,
    )
    def k(*refs):
        if with_hist:
            (table_hbm, src_hbm, dst_hbm, zeros_hbm, out_hbm,
             cnt_hbm, src_v, dst_v, rows_v, acc_s, gsems, ssems,
             hist_v, stage_s, red_v, csum_v) = refs
        else:
            (table_hbm, src_hbm, dst_hbm, zeros_hbm, out_hbm,
             src_v, dst_v, rows_v, acc_s, gsems, ssems) = refs
        c = lax.axis_index("c")
        s = lax.axis_index("s")
        wid = s * NC + c

        pltpu.sync_copy(src_hbm.at[wid], src_v)
        pltpu.sync_copy(dst_hbm.at[wid], dst_v)

        pltpu.sync_copy(zeros_hbm.at[pl.ds(s * rpt, rpt)],
                        acc_s.at[pl.ds(s * rpt, rpt)])

        if with_hist:
            def zero_hist(i, _):
                hist_v[pl.ds(i * 16, 16)] = jnp.zeros((16,), jnp.float32)
                return 0

            lax.fori_loop(0, N_PAD // 16, zero_hist, 0)

        plsc.subcore_barrier()

        for r in range(GLA):
            pltpu.async_copy(table_hbm.at[src_v.at[r]], rows_v[r], gsems[r])

        ones16 = jnp.ones((16,), jnp.float32)

        def body(i, _):
            for r in range(NBUF):
                j = i * NBUF + r
                pltpu.make_async_copy(
                    table_hbm.at[src_v.at[j]], rows_v[r], gsems[r]).wait()
                pltpu.async_copy(rows_v[r], acc_s.at[dst_v.at[j]], ssems[r],
                                 add=True)
                if with_hist:
                    for q in range(CHUNK // 16):
                        idx = dst_v[j, pl.ds(q * 16, 16)]
                        plsc.addupdate_scatter(hist_v, [idx], ones16)

                p = j + GLA
                rp = (r + GLA) % NBUF

                @pl.when((p < K) & (p >= NBUF))
                def _():
                    pltpu.make_async_copy(
                        rows_v[rp], acc_s.at[dst_v.at[p - NBUF]],
                        ssems[rp]).wait()

                @pl.when(p < K)
                def _():
                    pltpu.async_copy(
                        table_hbm.at[src_v.at[p]], rows_v[rp], gsems[rp])
            return 0

        lax.fori_loop(0, K // NBUF, body, 0)

        for i in range(NBUF):
            jj = K - NBUF + i
            pltpu.make_async_copy(
                rows_v[jj % NBUF], acc_s.at[dst_v.at[jj]],
                ssems[jj % NBUF]).wait()
        if with_hist:
            pltpu.sync_copy(hist_v, stage_s.at[s])
        plsc.subcore_barrier()

        pltpu.sync_copy(acc_s.at[pl.ds(s * rpt, rpt)],
                        out_hbm.at[c, pl.ds(s * rpt, rpt)])

        if with_hist:
            for r in range(NS):
                pltpu.sync_copy(stage_s.at[r, pl.ds(s * rpt, rpt)],
                                red_v.at[pl.ds(r * rpt, rpt)])

            def red_body(i, _):
                acc = red_v[pl.ds(i * 16, 16)]
                for r in range(1, NS):
                    acc = acc + red_v[pl.ds(r * rpt + i * 16, 16)]
                csum_v[pl.ds(i * 16, 16)] = acc
                return 0

            lax.fori_loop(0, rpt // 16, red_body, 0)
            pltpu.sync_copy(csum_v, cnt_hbm.at[c, pl.ds(s * rpt, rpt)])

    return k(table, src, dst, zeros)


def _proj0_tc(xp, wl0t, wr0t):
    def body(x_ref, wl_ref, wr_ref, t0_ref, z0_ref):
        xv = x_ref[...]
        t0_ref[...] = jnp.dot(xv, wl_ref[...],
                              preferred_element_type=jnp.float32)
        z0_ref[...] = jnp.dot(xv, wr_ref[...],
                              preferred_element_type=jnp.float32)

    return pl.pallas_call(
        body,
        out_shape=(
            jax.ShapeDtypeStruct((N_PAD, H), jnp.float32),
            jax.ShapeDtypeStruct((N_PAD, H), jnp.float32),
        ),
    )(xp, wl0t, wr0t)


def _mid_tc(p0, c0, z0, bl0_row, wl1t_pad, wr1t_pad, bl1_row):
    def body(p_ref, c_ref, z0_ref, bl0_ref, wl_ref, wr_ref, bl1_ref,
             t1_ref, z1_ref):
        ssum = p_ref[0] + p_ref[1]
        cnt = jnp.maximum(c_ref[0] + c_ref[1], 1.0)
        agg = ssum * (1.0 / cnt)
        h = jnp.maximum(agg + bl0_ref[...] + z0_ref[...], 0.0)
        y = jnp.dot(h, wl_ref[...], preferred_element_type=jnp.float32)
        col = lax.broadcasted_iota(jnp.int32, (1, 16), 1)
        t1_ref[...] = y + jnp.where(col == C, 1.0, 0.0)
        z1_ref[...] = jnp.dot(h, wr_ref[...],
                              preferred_element_type=jnp.float32) + bl1_ref[...]

    return pl.pallas_call(
        body,
        out_shape=(
            jax.ShapeDtypeStruct((N_PAD, 16), jnp.float32),
            jax.ShapeDtypeStruct((N_PAD, 16), jnp.float32),
        ),
    )(p0, c0, z0, bl0_row, wl1t_pad, wr1t_pad, bl1_row)


def _final_tc(p1, z1):
    def body(p_ref, z1_ref, o_ref):
        ssum = p_ref[0] + p_ref[1]
        cnt = jnp.maximum(ssum[:, C:C + 1], 1.0)
        o = ssum * (1.0 / cnt) + z1_ref[...]
        col = lax.broadcasted_iota(jnp.int32, (1, 16), 1)
        msk = col < C
        om = jnp.where(msk, o, -jnp.inf)
        m = jnp.max(om, axis=1, keepdims=True)
        e = jnp.where(msk, jnp.exp(om - m), 0.0)
        lse = jnp.log(jnp.sum(e, axis=1, keepdims=True))
        o_ref[...] = o - m - lse

    return pl.pallas_call(
        body,
        out_shape=jax.ShapeDtypeStruct((N_PAD, 16), jnp.float32),
    )(p1, z1)


def _pad_edges(edge_index):
    src = edge_index[0]
    dst = edge_index[1]
    pad = E_PAD - E
    src = jnp.concatenate([src, jnp.zeros((pad,), jnp.int32)])
    dst = jnp.concatenate([dst, jnp.full((pad,), N, jnp.int32)])
    return (src.reshape(NW, K, CHUNK), dst.reshape(NW, K, CHUNK))


def kernel(x, edge_index_0, edge_index_1, Wl0, bl0, Wr0, Wl1, bl1, Wr1):
    xp = jnp.pad(x, ((0, N_PAD - N), (0, 0)))
    src0, dst0 = _pad_edges(edge_index_0)
    src1, dst1 = _pad_edges(edge_index_1)

    wl0t = Wl0.T
    wr0t = Wr0.T
    wl1t_pad = jnp.pad(Wl1.T, ((0, 0), (0, 16 - C)))
    wr1t_pad = jnp.pad(Wr1.T, ((0, 0), (0, 16 - C)))
    bl0_row = bl0.reshape(1, H)
    bl1_row = jnp.pad(bl1, (0, 16 - C)).reshape(1, 16)

    table0, z0 = _proj0_tc(xp, wl0t, wr0t)
    p0, c0 = _segment_sum_sc(table0, src0, dst0, H, True)
    table1, z1 = _mid_tc(p0, c0.reshape(NC, N_PAD, 1), z0, bl0_row,
                         wl1t_pad, wr1t_pad, bl1_row)
    p1, = _segment_sum_sc(table1, src1, dst1, 16, False)
    out = _final_tc(p1, z1)
    return out[:N, :C]


# speedup vs baseline: 19.1179x; 1.3489x over previous
import functools

import jax
import jax.numpy as jnp
from jax import lax
from jax.experimental import pallas as pl
from jax.experimental.pallas import tpu as pltpu
from jax.experimental.pallas import tpu_sc as plsc

N = 10000
E = 320000
D = 128
H = 16
C = 14

NC = 2
NS = 16
CHUNK = 128
TCH = E // CHUNK
N_PAD = 10240
NBUF = 4


def _segment_sum_sc(table, edges, w, with_hist, k0, k1):
    rpt = N_PAD // NS
    left = TCH - NS * (k0 + k1)
    assert 0 <= left <= NS and k0 % NBUF == 0 and k1 % NBUF == 0

    mesh = plsc.VectorSubcoreMesh(core_axis_name="c", subcore_axis_name="s")

    out_type = [jax.ShapeDtypeStruct((NC, N_PAD, w), jnp.float32)]
    scratch = [
        pltpu.VMEM((k0, CHUNK), jnp.int32),
        pltpu.VMEM((k0, CHUNK), jnp.int32),
        pltpu.VMEM((1, CHUNK), jnp.int32),
        pltpu.VMEM((1, CHUNK), jnp.int32),
        [pltpu.VMEM((CHUNK, w), jnp.float32) for _ in range(NBUF)],
        pltpu.VMEM_SHARED((N_PAD, w), jnp.float32),
        [pltpu.SemaphoreType.DMA for _ in range(NBUF)],
    ]
    if with_hist:
        out_type.append(jax.ShapeDtypeStruct((NC, N_PAD), jnp.float32))
        scratch += [
            pltpu.VMEM((N_PAD,), jnp.float32),
            pltpu.VMEM_SHARED((NS, N_PAD), jnp.float32),
            pltpu.VMEM((NS * rpt,), jnp.float32),
            pltpu.VMEM((rpt,), jnp.float32),
        ]
    zeros = jnp.zeros((N_PAD, w), jnp.float32)

    @functools.partial(
        pl.kernel,
        mesh=mesh,
        compiler_params=pltpu.CompilerParams(
            use_tc_tiling_on_sc=False,
            needs_layout_passes=not with_hist,
        ),
        out_type=out_type,
        scratch_types=scratch,
    )
    def k(*refs):
        if with_hist:
            (table_hbm, edges_hbm, zeros_hbm, out_hbm, cnt_hbm,
             src_v, dst_v, srcx_v, dstx_v, rows_v, acc_s, sems,
             hist_v, stage_s, red_v, csum_v) = refs
        else:
            (table_hbm, edges_hbm, zeros_hbm, out_hbm,
             src_v, dst_v, srcx_v, dstx_v, rows_v, acc_s, sems) = refs
        c = lax.axis_index("c")
        s = lax.axis_index("s")
        kw = jnp.where(c == 0, k0, k1)

        @pl.when(c == 0)
        def _():
            pltpu.sync_copy(edges_hbm.at[0, pl.ds(s * k0, k0)],
                            src_v.at[pl.ds(0, k0)])
            pltpu.sync_copy(edges_hbm.at[1, pl.ds(s * k0, k0)],
                            dst_v.at[pl.ds(0, k0)])

        @pl.when(c == 1)
        def _():
            base = NS * k0 + s * k1
            pltpu.sync_copy(edges_hbm.at[0, pl.ds(base, k1)],
                            src_v.at[pl.ds(0, k1)])
            pltpu.sync_copy(edges_hbm.at[1, pl.ds(base, k1)],
                            dst_v.at[pl.ds(0, k1)])

        pltpu.sync_copy(zeros_hbm.at[pl.ds(s * rpt, rpt)],
                        acc_s.at[pl.ds(s * rpt, rpt)])

        if with_hist:
            def zero_hist(i, _):
                hist_v[pl.ds(i * 16, 16)] = jnp.zeros((16,), jnp.float32)
                return 0

            lax.fori_loop(0, N_PAD // 16, zero_hist, 0)

        plsc.subcore_barrier()

        ones16 = jnp.ones((16,), jnp.float32)

        @pl.when((c == 0) & (s < left))
        def _():
            lb = NS * (k0 + k1) + s
            pltpu.sync_copy(edges_hbm.at[0, pl.ds(lb, 1)], srcx_v)
            pltpu.sync_copy(edges_hbm.at[1, pl.ds(lb, 1)], dstx_v)
            pltpu.async_copy(
                table_hbm.at[srcx_v.at[0]], rows_v[0], sems[0]).wait()
            pltpu.sync_copy(rows_v[0], acc_s.at[dstx_v.at[0]], add=True)
            if with_hist:
                for q in range(CHUNK // 16):
                    idx = dstx_v[0, pl.ds(q * 16, 16)]
                    plsc.addupdate_scatter(hist_v, [idx], ones16)

        for r in range(NBUF):
            pltpu.async_copy(table_hbm.at[src_v.at[r]], rows_v[r], sems[r])

        def body(i, _):
            for r in range(NBUF):
                j = i * NBUF + r
                pltpu.make_async_copy(
                    table_hbm.at[src_v.at[j]], rows_v[r], sems[r]).wait()
                pltpu.sync_copy(rows_v[r], acc_s.at[dst_v.at[j]], add=True)
                if with_hist:
                    for q in range(CHUNK // 16):
                        idx = dst_v[j, pl.ds(q * 16, 16)]
                        plsc.addupdate_scatter(hist_v, [idx], ones16)

                @pl.when(j + NBUF < kw)
                def _():
                    pltpu.async_copy(
                        table_hbm.at[src_v.at[j + NBUF]], rows_v[r], sems[r])
            return 0

        lax.fori_loop(0, kw // NBUF, body, 0)

        if with_hist:
            pltpu.sync_copy(hist_v, stage_s.at[s])
        plsc.subcore_barrier()

        pltpu.sync_copy(acc_s.at[pl.ds(s * rpt, rpt)],
                        out_hbm.at[c, pl.ds(s * rpt, rpt)])

        if with_hist:
            for r in range(NS):
                pltpu.sync_copy(stage_s.at[r, pl.ds(s * rpt, rpt)],
                                red_v.at[pl.ds(r * rpt, rpt)])

            def red_body(i, _):
                acc = red_v[pl.ds(i * 16, 16)]
                for r in range(1, NS):
                    acc = acc + red_v[pl.ds(r * rpt + i * 16, 16)]
                csum_v[pl.ds(i * 16, 16)] = acc
                return 0

            lax.fori_loop(0, rpt // 16, red_body, 0)
            pltpu.sync_copy(csum_v, cnt_hbm.at[c, pl.ds(s * rpt, rpt)])

    return k(table, edges, zeros)


def _proj0_tc(x, wl0t, wr0t):
    def body(x_ref, wl_ref, wr_ref, t0_ref, z0_ref):
        xv = x_ref[...]
        t0_ref[0:N, :] = jnp.dot(xv, wl_ref[...],
                                 preferred_element_type=jnp.float32)
        z0_ref[0:N, :] = jnp.dot(xv, wr_ref[...],
                                 preferred_element_type=jnp.float32)

    return pl.pallas_call(
        body,
        out_shape=(
            jax.ShapeDtypeStruct((N_PAD, H), jnp.float32),
            jax.ShapeDtypeStruct((N_PAD, H), jnp.float32),
        ),
    )(x, wl0t, wr0t)


def _mid_tc(p0, c0, z0, bl0_row, wl1t_pad, wr1t_pad, bl1_row):
    def body(p_ref, c_ref, z0_ref, bl0_ref, wl_ref, wr_ref, bl1_ref,
             t1_ref, z1_ref):
        ssum = p_ref[0] + p_ref[1]
        cnt = jnp.maximum(c_ref[0] + c_ref[1], 1.0)
        agg = ssum * (1.0 / cnt)
        h = jnp.maximum(agg + bl0_ref[...] + z0_ref[...], 0.0)
        y = jnp.dot(h, wl_ref[...], preferred_element_type=jnp.float32)
        col = lax.broadcasted_iota(jnp.int32, (1, 16), 1)
        t1_ref[...] = y + jnp.where(col == C, 1.0, 0.0)
        z1_ref[...] = jnp.dot(h, wr_ref[...],
                              preferred_element_type=jnp.float32) + bl1_ref[...]

    return pl.pallas_call(
        body,
        out_shape=(
            jax.ShapeDtypeStruct((N_PAD, 16), jnp.float32),
            jax.ShapeDtypeStruct((N_PAD, 16), jnp.float32),
        ),
    )(p0, c0, z0, bl0_row, wl1t_pad, wr1t_pad, bl1_row)


def _final_tc(p1, z1):
    def body(p_ref, z1_ref, o_ref):
        ssum = p_ref[0] + p_ref[1]
        cnt = jnp.maximum(ssum[:, C:C + 1], 1.0)
        o = ssum * (1.0 / cnt) + z1_ref[...]
        col = lax.broadcasted_iota(jnp.int32, (1, 16), 1)
        msk = col < C
        om = jnp.where(msk, o, -jnp.inf)
        m = jnp.max(om, axis=1, keepdims=True)
        e = jnp.where(msk, jnp.exp(om - m), 0.0)
        lse = jnp.log(jnp.sum(e, axis=1, keepdims=True))
        o_ref[...] = o - m - lse

    return pl.pallas_call(
        body,
        out_shape=jax.ShapeDtypeStruct((N_PAD, 16), jnp.float32),
    )(p1, z1)


def kernel(x, edge_index_0, edge_index_1, Wl0, bl0, Wr0, Wl1, bl1, Wr1):
    e0 = edge_index_0.reshape(2, TCH, CHUNK)
    e1 = edge_index_1.reshape(2, TCH, CHUNK)

    wl0t = Wl0.T
    wr0t = Wr0.T
    wl1t_pad = jnp.pad(Wl1.T, ((0, 0), (0, 16 - C)))
    wr1t_pad = jnp.pad(Wr1.T, ((0, 0), (0, 16 - C)))
    bl0_row = bl0.reshape(1, H)
    bl1_row = jnp.pad(bl1, (0, 16 - C)).reshape(1, 16)

    table0, z0 = _proj0_tc(x, wl0t, wr0t)
    p0, c0 = _segment_sum_sc(table0, e0, H, True, 100, 56)
    table1, z1 = _mid_tc(p0, c0.reshape(NC, N_PAD, 1), z0, bl0_row,
                         wl1t_pad, wr1t_pad, bl1_row)
    p1, = _segment_sum_sc(table1, e1, 16, False, 112, 44)
    out = _final_tc(p1, z1)
    return out[:N, :C]


# speedup vs baseline: 22.1538x; 1.1588x over previous
import functools

import jax
import jax.numpy as jnp
from jax import lax
from jax.experimental import pallas as pl
from jax.experimental.pallas import tpu as pltpu
from jax.experimental.pallas import tpu_sc as plsc

N = 10000
E = 320000
D = 128
H = 16
C = 14

NC = 2
NS = 16
CHUNK = 128
TCH = E // CHUNK
N_PAD = 10240
G = N_PAD // 8
GN = N // 8
NBUF = 4


def _segment_sum_sc(table, src, dst, k0, k1):
    w = 16
    rpt = N_PAD // NS
    left = TCH - NS * (k0 + k1)
    assert 0 <= left <= NS and k0 % NBUF == 0 and k1 % NBUF == 0

    mesh = plsc.VectorSubcoreMesh(core_axis_name="c", subcore_axis_name="s")

    out_type = [
        jax.ShapeDtypeStruct((NC, N_PAD, w), jnp.float32),
        jax.ShapeDtypeStruct((NC, N_PAD, w), jnp.float32),
    ]
    scratch = [
        pltpu.VMEM((k0, CHUNK), jnp.int32),
        pltpu.VMEM((k0, CHUNK), jnp.int32),
        pltpu.VMEM((1, CHUNK), jnp.int32),
        pltpu.VMEM((1, CHUNK), jnp.int32),
        [pltpu.VMEM((CHUNK, w), jnp.float32) for _ in range(NBUF)],
        pltpu.VMEM_SHARED((N_PAD, w), jnp.float32),
        [pltpu.SemaphoreType.DMA for _ in range(NBUF)],
        pltpu.VMEM((N_PAD,), jnp.float32),
        pltpu.VMEM_SHARED((NS, N_PAD), jnp.float32),
        pltpu.VMEM((NS * rpt,), jnp.float32),
        pltpu.VMEM((rpt, w), jnp.float32),
    ]
    zeros = jnp.zeros((N_PAD, w), jnp.float32)

    @functools.partial(
        pl.kernel,
        mesh=mesh,
        compiler_params=pltpu.CompilerParams(
            use_tc_tiling_on_sc=False,
            needs_layout_passes=False,
        ),
        out_type=out_type,
        scratch_types=scratch,
    )
    def k(table_hbm, src_hbm, dst_hbm, zeros_hbm, out_hbm, cnt_hbm,
          src_v, dst_v, srcx_v, dstx_v, rows_v, acc_s, sems,
          hist_v, stage_s, red_v, cbc_v):
        c = lax.axis_index("c")
        s = lax.axis_index("s")
        kw = jnp.where(c == 0, k0, k1)

        @pl.when(c == 0)
        def _():
            pltpu.sync_copy(src_hbm.at[pl.ds(s * k0, k0)],
                            src_v.at[pl.ds(0, k0)])
            pltpu.sync_copy(dst_hbm.at[pl.ds(s * k0, k0)],
                            dst_v.at[pl.ds(0, k0)])

        @pl.when(c == 1)
        def _():
            base = NS * k0 + s * k1
            pltpu.sync_copy(src_hbm.at[pl.ds(base, k1)],
                            src_v.at[pl.ds(0, k1)])
            pltpu.sync_copy(dst_hbm.at[pl.ds(base, k1)],
                            dst_v.at[pl.ds(0, k1)])

        pltpu.sync_copy(zeros_hbm.at[pl.ds(s * rpt, rpt)],
                        acc_s.at[pl.ds(s * rpt, rpt)])

        def zero_hist(i, _):
            hist_v[pl.ds(i * 16, 16)] = jnp.zeros((16,), jnp.float32)
            return 0

        lax.fori_loop(0, N_PAD // 16, zero_hist, 0)

        plsc.subcore_barrier()

        ones16 = jnp.ones((16,), jnp.float32)

        @pl.when((c == 0) & (s < left))
        def _():
            lb = NS * (k0 + k1) + s
            pltpu.sync_copy(src_hbm.at[pl.ds(lb, 1)], srcx_v)
            pltpu.sync_copy(dst_hbm.at[pl.ds(lb, 1)], dstx_v)
            pltpu.async_copy(
                table_hbm.at[srcx_v.at[0]], rows_v[0], sems[0]).wait()
            pltpu.sync_copy(rows_v[0], acc_s.at[dstx_v.at[0]], add=True)
            for q in range(CHUNK // 16):
                idx = dstx_v[0, pl.ds(q * 16, 16)]
                plsc.addupdate_scatter(hist_v, [idx], ones16)

        for r in range(NBUF):
            pltpu.async_copy(table_hbm.at[src_v.at[r]], rows_v[r], sems[r])

        def body(i, _):
            for r in range(NBUF):
                j = i * NBUF + r
                pltpu.make_async_copy(
                    table_hbm.at[src_v.at[j]], rows_v[r], sems[r]).wait()
                pltpu.sync_copy(rows_v[r], acc_s.at[dst_v.at[j]], add=True)
                for q in range(CHUNK // 16):
                    idx = dst_v[j, pl.ds(q * 16, 16)]
                    plsc.addupdate_scatter(hist_v, [idx], ones16)

                @pl.when(j + NBUF < kw)
                def _():
                    pltpu.async_copy(
                        table_hbm.at[src_v.at[j + NBUF]], rows_v[r], sems[r])
            return 0

        lax.fori_loop(0, kw // NBUF, body, 0)
        pltpu.sync_copy(hist_v, stage_s.at[s])
        plsc.subcore_barrier()

        pltpu.sync_copy(acc_s.at[pl.ds(s * rpt, rpt)],
                        out_hbm.at[c, pl.ds(s * rpt, rpt)])

        for r in range(NS):
            pltpu.sync_copy(stage_s.at[r, pl.ds(s * rpt, rpt)],
                            red_v.at[pl.ds(r * rpt, rpt)])

        iota16 = lax.iota(jnp.int32, 16)

        def red_body(i, _):
            acc = red_v[pl.ds(i * 16, 16)]
            for r in range(1, NS):
                acc = acc + red_v[pl.ds(r * rpt + i * 16, 16)]
            rows_idx = iota16 + i * 16
            for col in range(16):
                plsc.store_scatter(
                    cbc_v, [rows_idx, jnp.full((16,), col, jnp.int32)], acc)
            return 0

        lax.fori_loop(0, rpt // 16, red_body, 0)
        pltpu.sync_copy(cbc_v, cnt_hbm.at[c, pl.ds(s * rpt, rpt)])

    return k(table, src, dst, zeros)


def _proj0_tc(x_pack, wbd_l, wbd_r):
    def body(x_ref, wl_ref, wr_ref, t0_ref, z0_ref):
        xv = x_ref[...]
        t0_ref[0:GN, :] = jnp.dot(xv, wl_ref[...],
                                  preferred_element_type=jnp.float32)
        z0_ref[0:GN, :] = jnp.dot(xv, wr_ref[...],
                                  preferred_element_type=jnp.float32)

    return pl.pallas_call(
        body,
        out_shape=(
            jax.ShapeDtypeStruct((G, 128), jnp.float32),
            jax.ShapeDtypeStruct((G, 128), jnp.float32),
        ),
    )(x_pack, wbd_l, wbd_r)


def _mid_tc(p0, cb0, z0_pack, bl0_tile, wbd1_l, wbd1_r, bl1_tile):
    def body(p_ref, c_ref, z0_ref, bl0_ref, wl_ref, wr_ref, bl1_ref,
             t1_ref, z1_ref):
        ssum = p_ref[0] + p_ref[1]
        cnt = jnp.maximum(c_ref[0] + c_ref[1], 1.0)
        h = jnp.maximum(ssum * (1.0 / cnt) + bl0_ref[...] + z0_ref[...], 0.0)
        t1_ref[...] = jnp.dot(h, wl_ref[...],
                              preferred_element_type=jnp.float32)
        z1_ref[...] = jnp.dot(h, wr_ref[...],
                              preferred_element_type=jnp.float32) + bl1_ref[...]

    return pl.pallas_call(
        body,
        out_shape=(
            jax.ShapeDtypeStruct((G, 128), jnp.float32),
            jax.ShapeDtypeStruct((G, 128), jnp.float32),
        ),
    )(p0, cb0, z0_pack, bl0_tile, wbd1_l, wbd1_r, bl1_tile)


def _final_tc(p1, cb1, z1_pack, mgrp):
    def body(p_ref, c_ref, z1_ref, m_ref, o_ref):
        ssum = p_ref[0] + p_ref[1]
        cnt = jnp.maximum(c_ref[0] + c_ref[1], 1.0)
        o = ssum * (1.0 / cnt) + z1_ref[...]
        col = lax.broadcasted_iota(jnp.int32, (1, 128), 1)
        e = jnp.where(col % 16 < C, jnp.exp(o), 0.0)
        gsum = jnp.dot(e, m_ref[...], preferred_element_type=jnp.float32)
        o_ref[...] = o - jnp.log(gsum)

    return pl.pallas_call(
        body,
        out_shape=jax.ShapeDtypeStruct((G, 128), jnp.float32),
    )(p1, cb1, z1_pack, mgrp)


def kernel(x, edge_index_0, edge_index_1, Wl0, bl0, Wr0, Wl1, bl1, Wr1):
    src0 = edge_index_0[0].reshape(TCH, CHUNK)
    dst0 = edge_index_0[1].reshape(TCH, CHUNK)
    src1 = edge_index_1[0].reshape(TCH, CHUNK)
    dst1 = edge_index_1[1].reshape(TCH, CHUNK)

    eye8 = jnp.eye(8, dtype=jnp.float32)
    wbd0_l = jnp.kron(eye8, Wl0.T)
    wbd0_r = jnp.kron(eye8, Wr0.T)
    wl1t_pad = jnp.pad(Wl1.T, ((0, 0), (0, 16 - C)))
    wr1t_pad = jnp.pad(Wr1.T, ((0, 0), (0, 16 - C)))
    wbd1_l = jnp.kron(eye8, wl1t_pad)
    wbd1_r = jnp.kron(eye8, wr1t_pad)
    bl0_tile = jnp.tile(bl0, 8).reshape(1, 128)
    bl1_tile = jnp.tile(jnp.pad(bl1, (0, 16 - C)), 8).reshape(1, 128)
    lane = jnp.arange(128)
    mgrp = (lane[:, None] // 16 == lane[None, :] // 16).astype(jnp.float32)

    x_pack = x.reshape(GN, 1024)

    t0_pack, z0_pack = _proj0_tc(x_pack, wbd0_l, wbd0_r)
    p0, cb0 = _segment_sum_sc(t0_pack.reshape(N_PAD, H), src0, dst0, 100, 56)
    t1_pack, z1_pack = _mid_tc(p0.reshape(NC, G, 128),
                               cb0.reshape(NC, G, 128),
                               z0_pack, bl0_tile, wbd1_l, wbd1_r, bl1_tile)
    p1, cb1 = _segment_sum_sc(t1_pack.reshape(N_PAD, 16), src1, dst1, 104, 52)
    out = _final_tc(p1.reshape(NC, G, 128), cb1.reshape(NC, G, 128),
                    z1_pack, mgrp)
    return out.reshape(N_PAD, 16)[:N, :C]


# speedup vs baseline: 24.6811x; 1.1141x over previous
import functools

import jax
import jax.numpy as jnp
from jax import lax
from jax.experimental import pallas as pl
from jax.experimental.pallas import tpu as pltpu
from jax.experimental.pallas import tpu_sc as plsc

N = 10000
E = 320000
D = 128
H = 16
C = 14

NC = 2
NS = 16
CHUNK = 128
TCH = E // CHUNK
N_PAD = 10240
G = N_PAD // 8
GN = N // 8
NBUF = 4


def _segment_sum_sc(table, src, dst, k0, k1, with_hist):
    w = 16
    rpt = N_PAD // NS
    left = TCH - NS * (k0 + k1)
    assert 0 <= left <= NS and k0 % NBUF == 0 and k1 % NBUF == 0

    mesh = plsc.VectorSubcoreMesh(core_axis_name="c", subcore_axis_name="s")

    out_type = [jax.ShapeDtypeStruct((NC, N_PAD, w), jnp.float32)]
    scratch = [
        pltpu.VMEM((k0, CHUNK), jnp.int32),
        pltpu.VMEM((k0, CHUNK), jnp.int32),
        pltpu.VMEM((1, CHUNK), jnp.int32),
        pltpu.VMEM((1, CHUNK), jnp.int32),
        [pltpu.VMEM((CHUNK, w), jnp.float32) for _ in range(NBUF)],
        pltpu.VMEM_SHARED((N_PAD, w), jnp.float32),
        [pltpu.SemaphoreType.DMA for _ in range(NBUF)],
    ]
    if with_hist:
        out_type.append(jax.ShapeDtypeStruct((NC, N_PAD, w), jnp.float32))
        scratch += [
            pltpu.VMEM((N_PAD,), jnp.float32),
            pltpu.VMEM_SHARED((NS, N_PAD), jnp.float32),
            pltpu.VMEM((NS * rpt,), jnp.float32),
            pltpu.VMEM((rpt, w), jnp.float32),
        ]
    zeros = jnp.zeros((N_PAD, w), jnp.float32)

    @functools.partial(
        pl.kernel,
        mesh=mesh,
        compiler_params=pltpu.CompilerParams(
            use_tc_tiling_on_sc=False,
            needs_layout_passes=False,
        ),
        out_type=out_type,
        scratch_types=scratch,
    )
    def k(*refs):
        if with_hist:
            (table_hbm, src_hbm, dst_hbm, zeros_hbm, out_hbm, cnt_hbm,
             src_v, dst_v, srcx_v, dstx_v, rows_v, acc_s, sems,
             hist_v, stage_s, red_v, cbc_v) = refs
        else:
            (table_hbm, src_hbm, dst_hbm, zeros_hbm, out_hbm,
             src_v, dst_v, srcx_v, dstx_v, rows_v, acc_s, sems) = refs
        c = lax.axis_index("c")
        s = lax.axis_index("s")
        kw = jnp.where(c == 0, k0, k1)

        @pl.when(c == 0)
        def _():
            pltpu.sync_copy(src_hbm.at[pl.ds(s * k0, k0)],
                            src_v.at[pl.ds(0, k0)])
            pltpu.sync_copy(dst_hbm.at[pl.ds(s * k0, k0)],
                            dst_v.at[pl.ds(0, k0)])

        @pl.when(c == 1)
        def _():
            base = NS * k0 + s * k1
            pltpu.sync_copy(src_hbm.at[pl.ds(base, k1)],
                            src_v.at[pl.ds(0, k1)])
            pltpu.sync_copy(dst_hbm.at[pl.ds(base, k1)],
                            dst_v.at[pl.ds(0, k1)])

        pltpu.sync_copy(zeros_hbm.at[pl.ds(s * rpt, rpt)],
                        acc_s.at[pl.ds(s * rpt, rpt)])

        if with_hist:
            def zero_hist(i, _):
                hist_v[pl.ds(i * 16, 16)] = jnp.zeros((16,), jnp.float32)
                return 0

            lax.fori_loop(0, N_PAD // 16, zero_hist, 0)

        plsc.subcore_barrier()

        ones16 = jnp.ones((16,), jnp.float32)

        @pl.when((c == 0) & (s < left))
        def _():
            lb = NS * (k0 + k1) + s
            pltpu.sync_copy(src_hbm.at[pl.ds(lb, 1)], srcx_v)
            pltpu.sync_copy(dst_hbm.at[pl.ds(lb, 1)], dstx_v)
            pltpu.async_copy(
                table_hbm.at[srcx_v.at[0]], rows_v[0], sems[0]).wait()
            pltpu.sync_copy(rows_v[0], acc_s.at[dstx_v.at[0]], add=True)
            if with_hist:
                for q in range(CHUNK // 16):
                    idx = dstx_v[0, pl.ds(q * 16, 16)]
                    plsc.addupdate_scatter(hist_v, [idx], ones16)

        for r in range(NBUF):
            pltpu.async_copy(table_hbm.at[src_v.at[r]], rows_v[r], sems[r])

        def body(i, _):
            for r in range(NBUF):
                j = i * NBUF + r
                pltpu.make_async_copy(
                    table_hbm.at[src_v.at[j]], rows_v[r], sems[r]).wait()
                pltpu.sync_copy(rows_v[r], acc_s.at[dst_v.at[j]], add=True)
                if with_hist:
                    for q in range(CHUNK // 16):
                        idx = dst_v[j, pl.ds(q * 16, 16)]
                        plsc.addupdate_scatter(hist_v, [idx], ones16)

                @pl.when(j + NBUF < kw)
                def _():
                    pltpu.async_copy(
                        table_hbm.at[src_v.at[j + NBUF]], rows_v[r], sems[r])
            return 0

        lax.fori_loop(0, kw // NBUF, body, 0)
        if with_hist:
            pltpu.sync_copy(hist_v, stage_s.at[s])
        plsc.subcore_barrier()

        pltpu.sync_copy(acc_s.at[pl.ds(s * rpt, rpt)],
                        out_hbm.at[c, pl.ds(s * rpt, rpt)])

        if with_hist:
            for r in range(NS):
                pltpu.sync_copy(stage_s.at[r, pl.ds(s * rpt, rpt)],
                                red_v.at[pl.ds(r * rpt, rpt)])

            iota16 = lax.iota(jnp.int32, 16)

            def red_body(i, _):
                acc = red_v[pl.ds(i * 16, 16)]
                for r in range(1, NS):
                    acc = acc + red_v[pl.ds(r * rpt + i * 16, 16)]
                rows_idx = iota16 + i * 16
                for col in range(16):
                    plsc.store_scatter(
                        cbc_v, [rows_idx, jnp.full((16,), col, jnp.int32)],
                        acc)
                return 0

            lax.fori_loop(0, rpt // 16, red_body, 0)
            pltpu.sync_copy(cbc_v, cnt_hbm.at[c, pl.ds(s * rpt, rpt)])

    return k(table, src, dst, zeros)


def _proj0_tc(x_pack, wbd_l, wbd_r):
    def body(x_ref, wl_ref, wr_ref, t0_ref, z0_ref):
        xv = x_ref[...]
        t0_ref[0:GN, :] = jnp.dot(xv, wl_ref[...],
                                  preferred_element_type=jnp.float32)
        z0_ref[0:GN, :] = jnp.dot(xv, wr_ref[...],
                                  preferred_element_type=jnp.float32)

    return pl.pallas_call(
        body,
        out_shape=(
            jax.ShapeDtypeStruct((G, 128), jnp.float32),
            jax.ShapeDtypeStruct((G, 128), jnp.float32),
        ),
    )(x_pack, wbd_l, wbd_r)


def _mid_tc(p0, cb0, z0_pack, bl0_tile, wbd1_l, wbd1_r, bl1_tile):
    def body(p_ref, c_ref, z0_ref, bl0_ref, wl_ref, wr_ref, bl1_ref,
             t1_ref, z1_ref):
        ssum = p_ref[0] + p_ref[1]
        cnt = jnp.maximum(c_ref[0] + c_ref[1], 1.0)
        h = jnp.maximum(ssum * (1.0 / cnt) + bl0_ref[...] + z0_ref[...], 0.0)
        col = lax.broadcasted_iota(jnp.int32, (1, 128), 1)
        t1_ref[...] = jnp.dot(h, wl_ref[...],
                              preferred_element_type=jnp.float32
                              ) + jnp.where(col % 16 == C, 1.0, 0.0)
        z1_ref[...] = jnp.dot(h, wr_ref[...],
                              preferred_element_type=jnp.float32) + bl1_ref[...]

    return pl.pallas_call(
        body,
        out_shape=(
            jax.ShapeDtypeStruct((G, 128), jnp.float32),
            jax.ShapeDtypeStruct((G, 128), jnp.float32),
        ),
    )(p0, cb0, z0_pack, bl0_tile, wbd1_l, wbd1_r, bl1_tile)


def _final_tc(p1, z1_pack, mgrp, m14):
    def body(p_ref, z1_ref, m_ref, m14_ref, o_ref):
        ssum = p_ref[0] + p_ref[1]
        cntb = jnp.dot(ssum, m14_ref[...], preferred_element_type=jnp.float32)
        o = ssum * (1.0 / jnp.maximum(cntb, 1.0)) + z1_ref[...]
        col = lax.broadcasted_iota(jnp.int32, (1, 128), 1)
        e = jnp.where(col % 16 < C, jnp.exp(o), 0.0)
        gsum = jnp.dot(e, m_ref[...], preferred_element_type=jnp.float32)
        o_ref[...] = o - jnp.log(gsum)

    return pl.pallas_call(
        body,
        out_shape=jax.ShapeDtypeStruct((G, 128), jnp.float32),
    )(p1, z1_pack, mgrp, m14)


def kernel(x, edge_index_0, edge_index_1, Wl0, bl0, Wr0, Wl1, bl1, Wr1):
    src0 = edge_index_0[0].reshape(TCH, CHUNK)
    dst0 = edge_index_0[1].reshape(TCH, CHUNK)
    src1 = edge_index_1[0].reshape(TCH, CHUNK)
    dst1 = edge_index_1[1].reshape(TCH, CHUNK)

    eye8 = jnp.eye(8, dtype=jnp.float32)
    wbd0_l = jnp.kron(eye8, Wl0.T)
    wbd0_r = jnp.kron(eye8, Wr0.T)
    wl1t_pad = jnp.pad(Wl1.T, ((0, 0), (0, 16 - C)))
    wr1t_pad = jnp.pad(Wr1.T, ((0, 0), (0, 16 - C)))
    wbd1_l = jnp.kron(eye8, wl1t_pad)
    wbd1_r = jnp.kron(eye8, wr1t_pad)
    bl0_tile = jnp.tile(bl0, 8).reshape(1, 128)
    bl1_tile = jnp.tile(jnp.pad(bl1, (0, 16 - C)), 8).reshape(1, 128)
    lane = jnp.arange(128)
    same_grp = lane[:, None] // 16 == lane[None, :] // 16
    mgrp = same_grp.astype(jnp.float32)
    m14 = (same_grp & (lane[:, None] % 16 == C)).astype(jnp.float32)

    x_pack = x.reshape(GN, 1024)

    t0_pack, z0_pack = _proj0_tc(x_pack, wbd0_l, wbd0_r)
    p0, cb0 = _segment_sum_sc(t0_pack.reshape(N_PAD, H), src0, dst0,
                              88, 68, True)
    t1_pack, z1_pack = _mid_tc(p0.reshape(NC, G, 128),
                               cb0.reshape(NC, G, 128),
                               z0_pack, bl0_tile, wbd1_l, wbd1_r, bl1_tile)
    p1, = _segment_sum_sc(t1_pack.reshape(N_PAD, 16), src1, dst1,
                          88, 68, False)
    out = _final_tc(p1.reshape(NC, G, 128), z1_pack, mgrp, m14)
    return out.reshape(N_PAD, 16)[:N, :C]


# speedup vs baseline: 24.9696x; 1.0117x over previous
import functools

import jax
import jax.numpy as jnp
from jax import lax
from jax.experimental import pallas as pl
from jax.experimental.pallas import tpu as pltpu
from jax.experimental.pallas import tpu_sc as plsc

N = 10000
E = 320000
D = 128
H = 16
C = 14

NC = 2
NS = 16
CHUNK = 128
TCH = E // CHUNK
N_PAD = 10240
G = N_PAD // 8
GN = N // 8
NBUF = 4


def _segment_sum_sc(table, src, dst, k0, k1, with_hist):
    w = 16
    rpt = N_PAD // NS
    left = TCH - NS * (k0 + k1)
    assert 0 <= left <= NS and k0 % NBUF == 0 and k1 % NBUF == 0

    mesh = plsc.VectorSubcoreMesh(core_axis_name="c", subcore_axis_name="s")

    out_type = [jax.ShapeDtypeStruct((NC, N_PAD, w), jnp.float32)]
    scratch = [
        pltpu.VMEM((k0, CHUNK), jnp.int32),
        pltpu.VMEM((k0, CHUNK), jnp.int32),
        pltpu.VMEM((1, CHUNK), jnp.int32),
        pltpu.VMEM((1, CHUNK), jnp.int32),
        [pltpu.VMEM((CHUNK, w), jnp.float32) for _ in range(NBUF)],
        pltpu.VMEM_SHARED((N_PAD, w), jnp.float32),
        [pltpu.SemaphoreType.DMA for _ in range(NBUF)],
    ]
    if with_hist:
        out_type.append(jax.ShapeDtypeStruct((NC, N_PAD, w), jnp.float32))
        scratch += [
            pltpu.VMEM((N_PAD,), jnp.float32),
            pltpu.VMEM_SHARED((NS, N_PAD), jnp.float32),
            pltpu.VMEM((NS * rpt,), jnp.float32),
            pltpu.VMEM((rpt, w), jnp.float32),
        ]
    zeros = jnp.zeros((N_PAD, w), jnp.float32)

    @functools.partial(
        pl.kernel,
        mesh=mesh,
        compiler_params=pltpu.CompilerParams(
            use_tc_tiling_on_sc=False,
            needs_layout_passes=False,
        ),
        out_type=out_type,
        scratch_types=scratch,
    )
    def k(*refs):
        if with_hist:
            (table_hbm, src_hbm, dst_hbm, zeros_hbm, out_hbm, cnt_hbm,
             src_v, dst_v, srcx_v, dstx_v, rows_v, acc_s, sems,
             hist_v, stage_s, red_v, cbc_v) = refs
        else:
            (table_hbm, src_hbm, dst_hbm, zeros_hbm, out_hbm,
             src_v, dst_v, srcx_v, dstx_v, rows_v, acc_s, sems) = refs
        c = lax.axis_index("c")
        s = lax.axis_index("s")
        kw = jnp.where(c == 0, k0, k1)

        @pl.when(c == 0)
        def _():
            pltpu.sync_copy(src_hbm.at[pl.ds(s * k0, k0)],
                            src_v.at[pl.ds(0, k0)])
            pltpu.sync_copy(dst_hbm.at[pl.ds(s * k0, k0)],
                            dst_v.at[pl.ds(0, k0)])

        @pl.when(c == 1)
        def _():
            base = NS * k0 + s * k1
            pltpu.sync_copy(src_hbm.at[pl.ds(base, k1)],
                            src_v.at[pl.ds(0, k1)])
            pltpu.sync_copy(dst_hbm.at[pl.ds(base, k1)],
                            dst_v.at[pl.ds(0, k1)])

        pltpu.sync_copy(zeros_hbm.at[pl.ds(s * rpt, rpt)],
                        acc_s.at[pl.ds(s * rpt, rpt)])

        if with_hist:
            def zero_hist(i, _):
                hist_v[pl.ds(i * 16, 16)] = jnp.zeros((16,), jnp.float32)
                return 0

            lax.fori_loop(0, N_PAD // 16, zero_hist, 0)

            def hist_body(j, _):
                for q in range(CHUNK // 16):
                    idx = dst_v[j, pl.ds(q * 16, 16)]
                    plsc.addupdate_scatter(hist_v, [idx],
                                           jnp.ones((16,), jnp.float32))
                return 0

            lax.fori_loop(0, kw, hist_body, 0)

        plsc.subcore_barrier()

        ones16 = jnp.ones((16,), jnp.float32)

        @pl.when((c == 0) & (s < left))
        def _():
            lb = NS * (k0 + k1) + s
            pltpu.sync_copy(src_hbm.at[pl.ds(lb, 1)], srcx_v)
            pltpu.sync_copy(dst_hbm.at[pl.ds(lb, 1)], dstx_v)
            pltpu.async_copy(
                table_hbm.at[srcx_v.at[0]], rows_v[0], sems[0]).wait()
            pltpu.sync_copy(rows_v[0], acc_s.at[dstx_v.at[0]], add=True)
            if with_hist:
                for q in range(CHUNK // 16):
                    idx = dstx_v[0, pl.ds(q * 16, 16)]
                    plsc.addupdate_scatter(hist_v, [idx], ones16)

        for r in range(NBUF):
            pltpu.async_copy(table_hbm.at[src_v.at[r]], rows_v[r], sems[r])

        def body(i, _):
            for r in range(NBUF):
                j = i * NBUF + r
                pltpu.make_async_copy(
                    table_hbm.at[src_v.at[j]], rows_v[r], sems[r]).wait()
                pltpu.sync_copy(rows_v[r], acc_s.at[dst_v.at[j]], add=True)

                @pl.when(j + NBUF < kw)
                def _():
                    pltpu.async_copy(
                        table_hbm.at[src_v.at[j + NBUF]], rows_v[r], sems[r])
            return 0

        lax.fori_loop(0, kw // NBUF, body, 0)
        if with_hist:
            pltpu.sync_copy(hist_v, stage_s.at[s])
        plsc.subcore_barrier()

        pltpu.sync_copy(acc_s.at[pl.ds(s * rpt, rpt)],
                        out_hbm.at[c, pl.ds(s * rpt, rpt)])

        if with_hist:
            for r in range(NS):
                pltpu.sync_copy(stage_s.at[r, pl.ds(s * rpt, rpt)],
                                red_v.at[pl.ds(r * rpt, rpt)])

            iota16 = lax.iota(jnp.int32, 16)

            def red_body(i, _):
                acc = red_v[pl.ds(i * 16, 16)]
                for r in range(1, NS):
                    acc = acc + red_v[pl.ds(r * rpt + i * 16, 16)]
                rows_idx = iota16 + i * 16
                for col in range(16):
                    plsc.store_scatter(
                        cbc_v, [rows_idx, jnp.full((16,), col, jnp.int32)],
                        acc)
                return 0

            lax.fori_loop(0, rpt // 16, red_body, 0)
            pltpu.sync_copy(cbc_v, cnt_hbm.at[c, pl.ds(s * rpt, rpt)])

    return k(table, src, dst, zeros)


def _proj0_tc(x_pack, wbd_l, wbd_r):
    def body(x_ref, wl_ref, wr_ref, t0_ref, z0_ref):
        xv = x_ref[...]
        t0_ref[0:GN, :] = jnp.dot(xv, wl_ref[...],
                                  preferred_element_type=jnp.float32)
        z0_ref[0:GN, :] = jnp.dot(xv, wr_ref[...],
                                  preferred_element_type=jnp.float32)

    return pl.pallas_call(
        body,
        out_shape=(
            jax.ShapeDtypeStruct((G, 128), jnp.float32),
            jax.ShapeDtypeStruct((G, 128), jnp.float32),
        ),
    )(x_pack, wbd_l, wbd_r)


def _mid_tc(p0, cb0, z0_pack, bl0_tile, wbd1_l, wbd1_r, bl1_tile):
    def body(p_ref, c_ref, z0_ref, bl0_ref, wl_ref, wr_ref, bl1_ref,
             t1_ref, z1_ref):
        ssum = p_ref[0] + p_ref[1]
        cnt = jnp.maximum(c_ref[0] + c_ref[1], 1.0)
        h = jnp.maximum(ssum * (1.0 / cnt) + bl0_ref[...] + z0_ref[...], 0.0)
        col = lax.broadcasted_iota(jnp.int32, (1, 128), 1)
        t1_ref[...] = jnp.dot(h, wl_ref[...],
                              preferred_element_type=jnp.float32
                              ) + jnp.where(col % 16 == C, 1.0, 0.0)
        z1_ref[...] = jnp.dot(h, wr_ref[...],
                              preferred_element_type=jnp.float32) + bl1_ref[...]

    return pl.pallas_call(
        body,
        out_shape=(
            jax.ShapeDtypeStruct((G, 128), jnp.float32),
            jax.ShapeDtypeStruct((G, 128), jnp.float32),
        ),
    )(p0, cb0, z0_pack, bl0_tile, wbd1_l, wbd1_r, bl1_tile)


def _final_tc(p1, z1_pack, mgrp, m14):
    def body(p_ref, z1_ref, m_ref, m14_ref, o_ref):
        ssum = p_ref[0] + p_ref[1]
        cntb = jnp.dot(ssum, m14_ref[...], preferred_element_type=jnp.float32)
        o = ssum * (1.0 / jnp.maximum(cntb, 1.0)) + z1_ref[...]
        col = lax.broadcasted_iota(jnp.int32, (1, 128), 1)
        e = jnp.where(col % 16 < C, jnp.exp(o), 0.0)
        gsum = jnp.dot(e, m_ref[...], preferred_element_type=jnp.float32)
        o_ref[...] = o - jnp.log(gsum)

    return pl.pallas_call(
        body,
        out_shape=jax.ShapeDtypeStruct((G, 128), jnp.float32),
    )(p1, z1_pack, mgrp, m14)


def kernel(x, edge_index_0, edge_index_1, Wl0, bl0, Wr0, Wl1, bl1, Wr1):
    src0 = edge_index_0[0].reshape(TCH, CHUNK)
    dst0 = edge_index_0[1].reshape(TCH, CHUNK)
    src1 = edge_index_1[0].reshape(TCH, CHUNK)
    dst1 = edge_index_1[1].reshape(TCH, CHUNK)

    eye8 = jnp.eye(8, dtype=jnp.float32)
    wbd0_l = jnp.kron(eye8, Wl0.T)
    wbd0_r = jnp.kron(eye8, Wr0.T)
    wl1t_pad = jnp.pad(Wl1.T, ((0, 0), (0, 16 - C)))
    wr1t_pad = jnp.pad(Wr1.T, ((0, 0), (0, 16 - C)))
    wbd1_l = jnp.kron(eye8, wl1t_pad)
    wbd1_r = jnp.kron(eye8, wr1t_pad)
    bl0_tile = jnp.tile(bl0, 8).reshape(1, 128)
    bl1_tile = jnp.tile(jnp.pad(bl1, (0, 16 - C)), 8).reshape(1, 128)
    lane = jnp.arange(128)
    same_grp = lane[:, None] // 16 == lane[None, :] // 16
    mgrp = same_grp.astype(jnp.float32)
    m14 = (same_grp & (lane[:, None] % 16 == C)).astype(jnp.float32)

    x_pack = x.reshape(GN, 1024)

    t0_pack, z0_pack = _proj0_tc(x_pack, wbd0_l, wbd0_r)
    p0, cb0 = _segment_sum_sc(t0_pack.reshape(N_PAD, H), src0, dst0,
                              80, 76, True)
    t1_pack, z1_pack = _mid_tc(p0.reshape(NC, G, 128),
                               cb0.reshape(NC, G, 128),
                               z0_pack, bl0_tile, wbd1_l, wbd1_r, bl1_tile)
    p1, = _segment_sum_sc(t1_pack.reshape(N_PAD, 16), src1, dst1,
                          80, 76, False)
    out = _final_tc(p1.reshape(NC, G, 128), z1_pack, mgrp, m14)
    return out.reshape(N_PAD, 16)[:N, :C]


# speedup vs baseline: 26.8140x; 1.0739x over previous
import functools

import jax
import jax.numpy as jnp
from jax import lax
from jax.experimental import pallas as pl
from jax.experimental.pallas import tpu as pltpu
from jax.experimental.pallas import tpu_sc as plsc

N = 10000
E = 320000
D = 128
H = 16
C = 14

NC = 2
NS = 16
CHUNK = 128
TCH = E // CHUNK
N_PAD = 10240
G = N_PAD // 8
GN = N // 8
NBUF = 4


def _segment_sum_sc(table, src, dst, k0, k1, with_hist):
    w = 16
    rpt = N_PAD // NS
    left = TCH - NS * (k0 + k1)
    assert 0 <= left <= NS and k0 % NBUF == 0 and k1 % NBUF == 0

    mesh = plsc.VectorSubcoreMesh(core_axis_name="c", subcore_axis_name="s")

    out_type = [jax.ShapeDtypeStruct((NC, N_PAD, w), jnp.float32)]
    scratch = [
        pltpu.VMEM((k0, CHUNK), jnp.int32),
        pltpu.VMEM((k0, CHUNK), jnp.int32),
        pltpu.VMEM((1, CHUNK), jnp.int32),
        pltpu.VMEM((1, CHUNK), jnp.int32),
        [pltpu.VMEM((CHUNK, w), jnp.float32) for _ in range(NBUF)],
        pltpu.VMEM_SHARED((N_PAD, w), jnp.float32),
        [pltpu.SemaphoreType.DMA for _ in range(NBUF)],
    ]
    if with_hist:
        out_type.append(jax.ShapeDtypeStruct((NC, N_PAD, w), jnp.float32))
        scratch += [
            pltpu.VMEM((N_PAD,), jnp.float32),
            pltpu.VMEM_SHARED((NS, N_PAD), jnp.float32),
            pltpu.VMEM((NS, rpt), jnp.float32),
            pltpu.VMEM((rpt, w), jnp.float32),
        ]
    zeros = jnp.zeros((N_PAD, w), jnp.float32)

    @functools.partial(
        pl.kernel,
        mesh=mesh,
        compiler_params=pltpu.CompilerParams(
            use_tc_tiling_on_sc=False,
            needs_layout_passes=False,
        ),
        out_type=out_type,
        scratch_types=scratch,
    )
    def k(*refs):
        if with_hist:
            (table_hbm, src_hbm, dst_hbm, zeros_hbm, out_hbm, cnt_hbm,
             src_v, dst_v, srcx_v, dstx_v, rows_v, acc_s, sems,
             hist_v, stage_s, red_v, cbc_v) = refs
        else:
            (table_hbm, src_hbm, dst_hbm, zeros_hbm, out_hbm,
             src_v, dst_v, srcx_v, dstx_v, rows_v, acc_s, sems) = refs
        c = lax.axis_index("c")
        s = lax.axis_index("s")
        kw = jnp.where(c == 0, k0, k1)

        @pl.when(c == 0)
        def _():
            pltpu.sync_copy(src_hbm.at[pl.ds(s * k0, k0)],
                            src_v.at[pl.ds(0, k0)])
            pltpu.sync_copy(dst_hbm.at[pl.ds(s * k0, k0)],
                            dst_v.at[pl.ds(0, k0)])

        @pl.when(c == 1)
        def _():
            base = NS * k0 + s * k1
            pltpu.sync_copy(src_hbm.at[pl.ds(base, k1)],
                            src_v.at[pl.ds(0, k1)])
            pltpu.sync_copy(dst_hbm.at[pl.ds(base, k1)],
                            dst_v.at[pl.ds(0, k1)])

        pltpu.sync_copy(zeros_hbm.at[pl.ds(s * rpt, rpt)],
                        acc_s.at[pl.ds(s * rpt, rpt)])

        if with_hist:
            def zero_hist(i, _):
                for q in range(8):
                    hist_v[pl.ds(i * 128 + q * 16, 16)] = jnp.zeros(
                        (16,), jnp.float32)
                return 0

            lax.fori_loop(0, N_PAD // 128, zero_hist, 0)

            def hist_body(j, _):
                for q in range(CHUNK // 16):
                    idx = dst_v[j, pl.ds(q * 16, 16)]
                    plsc.addupdate_scatter(hist_v, [idx],
                                           jnp.ones((16,), jnp.float32))
                return 0

            lax.fori_loop(0, kw, hist_body, 0)

        plsc.subcore_barrier()

        ones16 = jnp.ones((16,), jnp.float32)

        @pl.when((c == 0) & (s < left))
        def _():
            lb = NS * (k0 + k1) + s
            pltpu.sync_copy(src_hbm.at[pl.ds(lb, 1)], srcx_v)
            pltpu.sync_copy(dst_hbm.at[pl.ds(lb, 1)], dstx_v)
            pltpu.async_copy(
                table_hbm.at[srcx_v.at[0]], rows_v[0], sems[0]).wait()
            pltpu.sync_copy(rows_v[0], acc_s.at[dstx_v.at[0]], add=True)
            if with_hist:
                for q in range(CHUNK // 16):
                    idx = dstx_v[0, pl.ds(q * 16, 16)]
                    plsc.addupdate_scatter(hist_v, [idx], ones16)

        for r in range(NBUF):
            pltpu.async_copy(table_hbm.at[src_v.at[r]], rows_v[r], sems[r])

        def body(i, _):
            for r in range(NBUF):
                j = i * NBUF + r
                pltpu.make_async_copy(
                    table_hbm.at[src_v.at[j]], rows_v[r], sems[r]).wait()
                pltpu.sync_copy(rows_v[r], acc_s.at[dst_v.at[j]], add=True)

                @pl.when(j + NBUF < kw)
                def _():
                    pltpu.async_copy(
                        table_hbm.at[src_v.at[j + NBUF]], rows_v[r], sems[r])
            return 0

        lax.fori_loop(0, kw // NBUF, body, 0)
        if with_hist:
            pltpu.sync_copy(hist_v, stage_s.at[s])
        plsc.subcore_barrier()

        pltpu.sync_copy(acc_s.at[pl.ds(s * rpt, rpt)],
                        out_hbm.at[c, pl.ds(s * rpt, rpt)])

        if with_hist:
            pltpu.sync_copy(stage_s.at[:, pl.ds(s * rpt, rpt)], red_v)

            iota16 = lax.iota(jnp.int32, 16)

            def red_body(i, _):
                acc = red_v[0, pl.ds(i * 16, 16)]
                for r in range(1, NS):
                    acc = acc + red_v[r, pl.ds(i * 16, 16)]
                rows_idx = iota16 + i * 16
                for col in range(16):
                    plsc.store_scatter(
                        cbc_v, [rows_idx, jnp.full((16,), col, jnp.int32)],
                        acc)
                return 0

            lax.fori_loop(0, rpt // 16, red_body, 0)
            pltpu.sync_copy(cbc_v, cnt_hbm.at[c, pl.ds(s * rpt, rpt)])

    return k(table, src, dst, zeros)


def _proj0_tc(x_pack, wl0t, wr0t):
    def body(x_ref, wl_ref, wr_ref, t0_ref, z0_ref):
        wl = wl_ref[...]
        wr = wr_ref[...]
        for b in range(8):
            xb = x_ref[0:GN, 128 * b:128 * (b + 1)]
            t0_ref[0:GN, 16 * b:16 * (b + 1)] = jnp.dot(
                xb, wl, preferred_element_type=jnp.float32)
            z0_ref[0:GN, 16 * b:16 * (b + 1)] = jnp.dot(
                xb, wr, preferred_element_type=jnp.float32)

    return pl.pallas_call(
        body,
        out_shape=(
            jax.ShapeDtypeStruct((G, 128), jnp.float32),
            jax.ShapeDtypeStruct((G, 128), jnp.float32),
        ),
    )(x_pack, wl0t, wr0t)


def _mid_tc(p0, cb0, z0_pack, bl0_tile, wbd1_l, wbd1_r, bl1_tile):
    def body(p_ref, c_ref, z0_ref, bl0_ref, wl_ref, wr_ref, bl1_ref,
             t1_ref, z1_ref):
        ssum = p_ref[0] + p_ref[1]
        cnt = jnp.maximum(c_ref[0] + c_ref[1], 1.0)
        h = jnp.maximum(ssum * (1.0 / cnt) + bl0_ref[...] + z0_ref[...], 0.0)
        col = lax.broadcasted_iota(jnp.int32, (1, 128), 1)
        t1_ref[...] = jnp.dot(h, wl_ref[...],
                              preferred_element_type=jnp.float32
                              ) + jnp.where(col % 16 == C, 1.0, 0.0)
        z1_ref[...] = jnp.dot(h, wr_ref[...],
                              preferred_element_type=jnp.float32) + bl1_ref[...]

    return pl.pallas_call(
        body,
        out_shape=(
            jax.ShapeDtypeStruct((G, 128), jnp.float32),
            jax.ShapeDtypeStruct((G, 128), jnp.float32),
        ),
    )(p0, cb0, z0_pack, bl0_tile, wbd1_l, wbd1_r, bl1_tile)


def _final_tc(p1, z1_pack, mgrp, m14):
    def body(p_ref, z1_ref, m_ref, m14_ref, o_ref):
        ssum = p_ref[0] + p_ref[1]
        cntb = jnp.dot(ssum, m14_ref[...], preferred_element_type=jnp.float32)
        o = ssum * (1.0 / jnp.maximum(cntb, 1.0)) + z1_ref[...]
        col = lax.broadcasted_iota(jnp.int32, (1, 128), 1)
        e = jnp.where(col % 16 < C, jnp.exp(o), 0.0)
        gsum = jnp.dot(e, m_ref[...], preferred_element_type=jnp.float32)
        o_ref[...] = o - jnp.log(gsum)

    return pl.pallas_call(
        body,
        out_shape=jax.ShapeDtypeStruct((G, 128), jnp.float32),
    )(p1, z1_pack, mgrp, m14)


def kernel(x, edge_index_0, edge_index_1, Wl0, bl0, Wr0, Wl1, bl1, Wr1):
    src0 = edge_index_0[0].reshape(TCH, CHUNK)
    dst0 = edge_index_0[1].reshape(TCH, CHUNK)
    src1 = edge_index_1[0].reshape(TCH, CHUNK)
    dst1 = edge_index_1[1].reshape(TCH, CHUNK)

    eye8 = jnp.eye(8, dtype=jnp.float32)
    wl1t_pad = jnp.pad(Wl1.T, ((0, 0), (0, 16 - C)))
    wr1t_pad = jnp.pad(Wr1.T, ((0, 0), (0, 16 - C)))
    wbd1_l = jnp.kron(eye8, wl1t_pad)
    wbd1_r = jnp.kron(eye8, wr1t_pad)
    bl0_tile = jnp.tile(bl0, 8).reshape(1, 128)
    bl1_tile = jnp.tile(jnp.pad(bl1, (0, 16 - C)), 8).reshape(1, 128)
    lane = jnp.arange(128)
    same_grp = lane[:, None] // 16 == lane[None, :] // 16
    mgrp = same_grp.astype(jnp.float32)
    m14 = (same_grp & (lane[:, None] % 16 == C)).astype(jnp.float32)

    x_pack = x.reshape(GN, 1024)

    t0_pack, z0_pack = _proj0_tc(x_pack, Wl0.T, Wr0.T)
    p0, cb0 = _segment_sum_sc(t0_pack.reshape(N_PAD, H), src0, dst0,
                              80, 76, True)
    t1_pack, z1_pack = _mid_tc(p0.reshape(NC, G, 128),
                               cb0.reshape(NC, G, 128),
                               z0_pack, bl0_tile, wbd1_l, wbd1_r, bl1_tile)
    p1, = _segment_sum_sc(t1_pack.reshape(N_PAD, 16), src1, dst1,
                          80, 76, False)
    out = _final_tc(p1.reshape(NC, G, 128), z1_pack, mgrp, m14)
    return out.reshape(N_PAD, 16)[:N, :C]
